# Initial kernel scaffold; baseline (speedup 1.0000x reference)
import jax
import jax.numpy as jnp
from jax.experimental import pallas as pl


def kernel(x, emb, W, b):
    raise NotImplementedError("write your pallas kernel here")


# speedup vs baseline: 119.2920x; 119.2920x over previous
import functools

import jax
import jax.numpy as jnp
from jax import lax
from jax.experimental import pallas as pl
from jax.experimental.pallas import tpu as pltpu
from jax.experimental.pallas import tpu_sc as plsc

V = 64
D = 16
LANES = 16


def _make_sc_histogram(B, L):
    info = plsc.get_sparse_core_info()
    NC, NS = info.num_cores, info.num_subcores
    NW = NC * NS
    rows_per_w = B // NW
    chunk = min(rows_per_w, 128)
    n_chunks = rows_per_w // chunk
    k_full = L // LANES
    rem = L % LANES

    mesh = plsc.VectorSubcoreMesh(core_axis_name="c", subcore_axis_name="s")

    @functools.partial(
        pl.kernel,
        mesh=mesh,
        compiler_params=pltpu.CompilerParams(needs_layout_passes=False),
        out_type=jax.ShapeDtypeStruct((B * V,), jnp.float32),
        scratch_types=[
            pltpu.VMEM((chunk, L), jnp.int32),
            pltpu.VMEM((chunk * V,), jnp.float32),
        ],
    )
    def hist(x_hbm, out_hbm, xbuf, cbuf):
        wid = lax.axis_index("s") * NC + lax.axis_index("c")
        base = wid * rows_per_w
        zeros16 = jnp.zeros((LANES,), jnp.float32)
        ones16 = jnp.full((LANES,), 1.0, jnp.float32)
        lane = lax.iota(jnp.int32, LANES)
        tailmask = lane >= (LANES - rem)

        for c in range(n_chunks):
            row0 = base + c * chunk
            pltpu.sync_copy(x_hbm.at[pl.ds(row0, chunk), :], xbuf)

            def row_body(r, carry):
                rbase = jnp.full((LANES,), r * V, jnp.int32)
                for v in range(V // LANES):
                    cbuf[pl.ds(r * V + v * LANES, LANES)] = zeros16
                for k in range(k_full):
                    tok = xbuf[r, pl.ds(k * LANES, LANES)]
                    plsc.addupdate_scatter(cbuf, [rbase + tok], ones16)
                if rem:
                    tok = xbuf[r, pl.ds(L - LANES, LANES)]
                    plsc.addupdate_scatter(cbuf, [rbase + tok], ones16,
                                           mask=tailmask)
                return carry

            lax.fori_loop(0, chunk, row_body, 0)
            pltpu.sync_copy(cbuf, out_hbm.at[pl.ds(row0 * V, chunk * V)])

    return hist


def _tc_head_body(inv_l, counts_ref, emb_ref, w_ref, b_ref, out_ref):
    m2 = lax.dot_general(emb_ref[...], w_ref[...],
                         (((1,), (1,)), ((), ())),
                         preferred_element_type=jnp.float32) * inv_l
    out_ref[...] = lax.dot_general(counts_ref[...], m2,
                                   (((1,), (0,)), ((), ())),
                                   preferred_element_type=jnp.float32) + b_ref[...]


def _tc_head(counts, emb, W, b2d, L):
    B = counts.shape[0]
    tb = min(B, 2048)
    grid = (B // tb,)
    return pl.pallas_call(
        functools.partial(_tc_head_body, 1.0 / L),
        grid=grid,
        in_specs=[
            pl.BlockSpec((tb, V), lambda i: (i, 0)),
            pl.BlockSpec((V, D), lambda i: (0, 0)),
            pl.BlockSpec((V, D), lambda i: (0, 0)),
            pl.BlockSpec((1, V), lambda i: (0, 0)),
        ],
        out_specs=pl.BlockSpec((tb, V), lambda i: (i, 0)),
        out_shape=jax.ShapeDtypeStruct((B, V), jnp.float32),
    )(counts, emb, W, b2d)


def kernel(x, emb, W, b):
    B, L = x.shape
    x = x.astype(jnp.int32)
    counts = _make_sc_histogram(B, L)(x).reshape(B, V)
    return _tc_head(counts, emb, W, b.reshape(1, V), L)


# speedup vs baseline: 169.1799x; 1.4182x over previous
import functools

import jax
import jax.numpy as jnp
from jax import lax
from jax.experimental import pallas as pl
from jax.experimental.pallas import tpu as pltpu
from jax.experimental.pallas import tpu_sc as plsc

V = 64
D = 16
LANES = 16


def _make_sc_histogram(B, L):
    info = plsc.get_sparse_core_info()
    NC, NS = info.num_cores, info.num_subcores
    NW = NC * NS
    half_rows_w = (B // 2) // NW
    chunkb = min(half_rows_w, 128)
    n_chunks = half_rows_w // chunkb
    subb = chunkb // 2
    n_groups = subb // LANES

    mesh = plsc.VectorSubcoreMesh(core_axis_name="c", subcore_axis_name="s")

    @functools.partial(
        pl.kernel,
        mesh=mesh,
        compiler_params=pltpu.CompilerParams(needs_layout_passes=False),
        out_type=jax.ShapeDtypeStruct(((B // 2) * 2 * V,), jnp.float32),
        scratch_types=[
            pltpu.VMEM((L, chunkb), jnp.int32),
            pltpu.VMEM((L, chunkb), jnp.int32),
            pltpu.VMEM((L, chunkb), jnp.int32),
            pltpu.VMEM((L, chunkb), jnp.int32),
            pltpu.VMEM((subb * 2 * V,), jnp.float32),
            pltpu.VMEM((subb * 2 * V,), jnp.float32),
            pltpu.SemaphoreType.DMA,
            pltpu.SemaphoreType.DMA,
            pltpu.SemaphoreType.DMA,
            pltpu.SemaphoreType.DMA,
        ],
    )
    def hist(xt_hbm, out_hbm, xa0, xa1, xb0, xb1, cbuf0, cbuf1,
             in0, in1, out0, out1):
        wid = lax.axis_index("s") * NC + lax.axis_index("c")
        base2 = wid * half_rows_w
        zeros16 = jnp.zeros((LANES,), jnp.float32)
        ones16 = jnp.full((LANES,), 1.0, jnp.float32)
        row_step = lax.iota(jnp.int32, LANES) * (2 * V)
        xabufs = (xa0, xa1)
        xbbufs = (xb0, xb1)
        cbufs = (cbuf0, cbuf1)
        in_sems = (in0, in1)
        out_sems = (out0, out1)

        def start_in(c):
            col0 = base2 + c * chunkb
            sem = in_sems[c % 2]
            return [
                pltpu.async_copy(
                    xt_hbm.at[:, pl.ds(col0, chunkb)], xabufs[c % 2], sem),
                pltpu.async_copy(
                    xt_hbm.at[:, pl.ds(col0 + B // 2, chunkb)],
                    xbbufs[c % 2], sem),
            ]

        def start_out(s, sl):
            row0 = base2 + s * subb
            return pltpu.async_copy(
                cbufs[sl],
                out_hbm.at[pl.ds(row0 * 2 * V, subb * 2 * V)],
                out_sems[sl])

        pending_in = [start_in(c) for c in range(n_chunks)]
        pending_out = [None, None]
        for c in range(n_chunks):
            for h in pending_in[c]:
                h.wait()
            xa = xabufs[c % 2]
            xb = xbbufs[c % 2]
            for sub in range(chunkb // subb):
                s = c * (chunkb // subb) + sub
                sl = s % 2
                if pending_out[sl] is not None:
                    pending_out[sl].wait()
                    pending_out[sl] = None
                cb = cbufs[sl]

                @plsc.parallel_loop(0, subb * 2 * V // LANES, 1, unroll=8)
                def zero_body(r):
                    cb[pl.ds(r * LANES, LANES)] = zeros16

                def l_body(l, carry):
                    for p, xp in ((0, xa), (1, xb)):
                        for g in range(n_groups):
                            win = cb.at[pl.ds(g * LANES * 2 * V + p * V,
                                              (LANES - 1) * 2 * V + V)]
                            tok = xp[l, pl.ds(sub * subb + g * LANES, LANES)]
                            plsc.addupdate_scatter(win, [row_step + tok],
                                                   ones16)
                    return carry

                lax.fori_loop(0, L, l_body, 0)
                pending_out[sl] = start_out(s, sl)
        for p in pending_out:
            if p is not None:
                p.wait()

    return hist


def _tc_head_body(inv_l, counts_ref, emb_ref, w_ref, b_ref, out_ref):
    m2 = lax.dot_general(emb_ref[...], w_ref[...],
                         (((1,), (1,)), ((), ())),
                         preferred_element_type=jnp.float32) * inv_l
    h = pl.program_id(0)
    m2s = jnp.concatenate([m2, m2], axis=0)
    rid = lax.broadcasted_iota(jnp.int32, (2 * V, V), 0)
    m2h = jnp.where((rid // V) == h, m2s, 0.0)
    out_ref[...] = lax.dot_general(m2h, counts_ref[...],
                                   (((0,), (1,)), ((), ())),
                                   preferred_element_type=jnp.float32) + b_ref[...]


def _tc_head(counts2, emb, W, bcol, L):
    Bh = counts2.shape[0]
    tb = min(Bh, 2048)
    nb = Bh // tb
    return pl.pallas_call(
        functools.partial(_tc_head_body, 1.0 / L),
        grid=(2, nb),
        in_specs=[
            pl.BlockSpec((tb, 2 * V), lambda h, j: (j, 0)),
            pl.BlockSpec((V, D), lambda h, j: (0, 0)),
            pl.BlockSpec((V, D), lambda h, j: (0, 0)),
            pl.BlockSpec((V, 1), lambda h, j: (0, 0)),
        ],
        out_specs=pl.BlockSpec((V, tb), lambda h, j: (0, h * nb + j)),
        out_shape=jax.ShapeDtypeStruct((V, 2 * Bh), jnp.float32),
    )(counts2, emb, W, bcol)


def kernel(x, emb, W, b):
    B, L = x.shape
    x = x.astype(jnp.int32)
    counts2 = _make_sc_histogram(B, L)(x.T).reshape(B // 2, 2 * V)
    out_t = _tc_head(counts2, emb, W, b.reshape(V, 1), L)
    return out_t.T


# speedup vs baseline: 288.6208x; 1.7060x over previous
import functools

import jax
import jax.numpy as jnp
from jax import lax
from jax.experimental import pallas as pl
from jax.experimental.pallas import tpu as pltpu
from jax.experimental.pallas import tpu_sc as plsc

V = 64
D = 16
LANES = 16


def _make_sc_histogram(B, L):
    info = plsc.get_sparse_core_info()
    NC, NS = info.num_cores, info.num_subcores
    NW = NC * NS
    half_rows_w = (B // 2) // NW
    chunkb = min(half_rows_w, 128)
    n_chunks = half_rows_w // chunkb
    subb = chunkb // 2
    n_groups = subb // LANES

    mesh = plsc.VectorSubcoreMesh(core_axis_name="c", subcore_axis_name="s")

    @functools.partial(
        pl.kernel,
        mesh=mesh,
        compiler_params=pltpu.CompilerParams(needs_layout_passes=False),
        out_type=jax.ShapeDtypeStruct(((B // 2) * 2 * V,), jnp.float32),
        scratch_types=[
            pltpu.VMEM((L, chunkb), jnp.int32),
            pltpu.VMEM((L, chunkb), jnp.int32),
            pltpu.VMEM((L, chunkb), jnp.int32),
            pltpu.VMEM((L, chunkb), jnp.int32),
            pltpu.VMEM((subb * 2 * V,), jnp.float32),
            pltpu.VMEM((subb * 2 * V,), jnp.float32),
            pltpu.SemaphoreType.DMA,
            pltpu.SemaphoreType.DMA,
            pltpu.SemaphoreType.DMA,
            pltpu.SemaphoreType.DMA,
        ],
    )
    def hist(xt_hbm, out_hbm, xa0, xa1, xb0, xb1, cbuf0, cbuf1,
             in0, in1, out0, out1):
        wid = lax.axis_index("s") * NC + lax.axis_index("c")
        base2 = wid * half_rows_w
        zeros16 = jnp.zeros((LANES,), jnp.float32)
        ones16 = jnp.full((LANES,), 1.0, jnp.float32)
        row_step = lax.iota(jnp.int32, LANES) * (2 * V)
        xabufs = (xa0, xa1)
        xbbufs = (xb0, xb1)
        cbufs = (cbuf0, cbuf1)
        in_sems = (in0, in1)
        out_sems = (out0, out1)

        def start_in(c):
            col0 = base2 + c * chunkb
            sem = in_sems[c % 2]
            return [
                pltpu.async_copy(
                    xt_hbm.at[:, pl.ds(col0, chunkb)], xabufs[c % 2], sem),
                pltpu.async_copy(
                    xt_hbm.at[:, pl.ds(col0 + B // 2, chunkb)],
                    xbbufs[c % 2], sem),
            ]

        def start_out(s, sl):
            row0 = base2 + s * subb
            return pltpu.async_copy(
                cbufs[sl],
                out_hbm.at[pl.ds(row0 * 2 * V, subb * 2 * V)],
                out_sems[sl])

        pending_in = [start_in(c) for c in range(n_chunks)]
        pending_out = [None, None]
        for c in range(n_chunks):
            for h in pending_in[c]:
                h.wait()
            xa = xabufs[c % 2]
            xb = xbbufs[c % 2]
            for sub in range(chunkb // subb):
                s = c * (chunkb // subb) + sub
                sl = s % 2
                if pending_out[sl] is not None:
                    pending_out[sl].wait()
                    pending_out[sl] = None
                cb = cbufs[sl]

                @plsc.parallel_loop(0, subb * 2 * V // LANES, 1, unroll=8)
                def zero_body(r):
                    cb[pl.ds(r * LANES, LANES)] = zeros16

                def l_body(li, carry):
                    work = []
                    for u in range(2):
                        l = li * 2 + u
                        for p, xp in ((0, xa), (1, xb)):
                            for g in range(n_groups):
                                win = cb.at[pl.ds(
                                    g * LANES * 2 * V + p * V,
                                    (LANES - 1) * 2 * V + V)]
                                tok = xp[l, pl.ds(sub * subb + g * LANES,
                                                  LANES)]
                                work.append((win, tok))
                    idxs = [row_step + tok for _, tok in work]
                    for (win, _), idx in zip(work, idxs):
                        plsc.addupdate_scatter(win, [idx], ones16)
                    return carry

                lax.fori_loop(0, L // 2, l_body, 0)
                pending_out[sl] = start_out(s, sl)
        for p in pending_out:
            if p is not None:
                p.wait()

    return hist


def _tc_head_body(inv_l, counts_ref, emb_ref, w_ref, b_ref, out_ref):
    m2 = lax.dot_general(emb_ref[...], w_ref[...],
                         (((1,), (1,)), ((), ())),
                         preferred_element_type=jnp.float32) * inv_l
    h = pl.program_id(0)
    m2s = jnp.concatenate([m2, m2], axis=0)
    rid = lax.broadcasted_iota(jnp.int32, (2 * V, V), 0)
    m2h = jnp.where((rid // V) == h, m2s, 0.0)
    out_ref[...] = lax.dot_general(m2h, counts_ref[...],
                                   (((0,), (1,)), ((), ())),
                                   preferred_element_type=jnp.float32) + b_ref[...]


def _tc_head(counts2, emb, W, bcol, L):
    Bh = counts2.shape[0]
    tb = min(Bh, 2048)
    nb = Bh // tb
    return pl.pallas_call(
        functools.partial(_tc_head_body, 1.0 / L),
        grid=(2, nb),
        in_specs=[
            pl.BlockSpec((tb, 2 * V), lambda h, j: (j, 0)),
            pl.BlockSpec((V, D), lambda h, j: (0, 0)),
            pl.BlockSpec((V, D), lambda h, j: (0, 0)),
            pl.BlockSpec((V, 1), lambda h, j: (0, 0)),
        ],
        out_specs=pl.BlockSpec((V, tb), lambda h, j: (0, h * nb + j)),
        out_shape=jax.ShapeDtypeStruct((V, 2 * Bh), jnp.float32),
    )(counts2, emb, W, bcol)


def kernel(x, emb, W, b):
    B, L = x.shape
    x = x.astype(jnp.int32)
    counts2 = _make_sc_histogram(B, L)(x.T).reshape(B // 2, 2 * V)
    out_t = _tc_head(counts2, emb, W, b.reshape(V, 1), L)
    return out_t.T


# speedup vs baseline: 300.5674x; 1.0414x over previous
import functools

import jax
import jax.numpy as jnp
from jax import lax
from jax.experimental import pallas as pl
from jax.experimental.pallas import tpu as pltpu
from jax.experimental.pallas import tpu_sc as plsc

V = 64
D = 16
LANES = 16


def _make_sc_histogram(B, L):
    info = plsc.get_sparse_core_info()
    NC, NS = info.num_cores, info.num_subcores
    NW = NC * NS
    half_rows_w = (B // 2) // NW
    chunkb = min(half_rows_w, 128)
    n_chunks = half_rows_w // chunkb
    subb = chunkb // 2
    n_groups = subb // LANES

    mesh = plsc.VectorSubcoreMesh(core_axis_name="c", subcore_axis_name="s")

    @functools.partial(
        pl.kernel,
        mesh=mesh,
        compiler_params=pltpu.CompilerParams(needs_layout_passes=False),
        out_type=jax.ShapeDtypeStruct(((B // 2) * 2 * V,), jnp.float32),
        scratch_types=[
            pltpu.VMEM((L, chunkb), jnp.int32),
            pltpu.VMEM((L, chunkb), jnp.int32),
            pltpu.VMEM((L, chunkb), jnp.int32),
            pltpu.VMEM((L, chunkb), jnp.int32),
            pltpu.VMEM((subb * 2 * V,), jnp.float32),
            pltpu.VMEM((subb * 2 * V,), jnp.float32),
            pltpu.SemaphoreType.DMA,
            pltpu.SemaphoreType.DMA,
            pltpu.SemaphoreType.DMA,
            pltpu.SemaphoreType.DMA,
        ],
    )
    def hist(xt_hbm, out_hbm, xa0, xa1, xb0, xb1, cbuf0, cbuf1,
             in0, in1, out0, out1):
        wid = lax.axis_index("s") * NC + lax.axis_index("c")
        base2 = wid * half_rows_w
        zeros16 = jnp.zeros((LANES,), jnp.float32)
        ones16 = jnp.full((LANES,), 1.0, jnp.float32)
        row_step = lax.iota(jnp.int32, LANES) * (2 * V)
        xabufs = (xa0, xa1)
        xbbufs = (xb0, xb1)
        cbufs = (cbuf0, cbuf1)
        in_sems = (in0, in1)
        out_sems = (out0, out1)

        def start_in(c):
            col0 = base2 + c * chunkb
            sem = in_sems[c % 2]
            return [
                pltpu.async_copy(
                    xt_hbm.at[:, pl.ds(col0, chunkb)], xabufs[c % 2], sem),
                pltpu.async_copy(
                    xt_hbm.at[:, pl.ds(col0 + B // 2, chunkb)],
                    xbbufs[c % 2], sem),
            ]

        def start_out(s, sl):
            row0 = base2 + s * subb
            return pltpu.async_copy(
                cbufs[sl],
                out_hbm.at[pl.ds(row0 * 2 * V, subb * 2 * V)],
                out_sems[sl])

        pending_in = [start_in(c) for c in range(n_chunks)]
        pending_out = [None, None]
        for c in range(n_chunks):
            for h in pending_in[c]:
                h.wait()
            xa = xabufs[c % 2]
            xb = xbbufs[c % 2]
            for sub in range(chunkb // subb):
                s = c * (chunkb // subb) + sub
                sl = s % 2
                if pending_out[sl] is not None:
                    pending_out[sl].wait()
                    pending_out[sl] = None
                cb = cbufs[sl]

                @plsc.parallel_loop(0, subb * 2 * V // LANES, 1, unroll=8)
                def zero_body(r):
                    cb[pl.ds(r * LANES, LANES)] = zeros16

                def l_body(li, carry):
                    work = []
                    for u in range(4):
                        l = li * 4 + u
                        for p, xp in ((0, xa), (1, xb)):
                            for g in range(n_groups):
                                win = cb.at[pl.ds(
                                    g * LANES * 2 * V + p * V,
                                    (LANES - 1) * 2 * V + V)]
                                tok = xp[l, pl.ds(sub * subb + g * LANES,
                                                  LANES)]
                                work.append((win, tok))
                    idxs = [row_step + tok for _, tok in work]
                    for (win, _), idx in zip(work, idxs):
                        plsc.addupdate_scatter(win, [idx], ones16)
                    return carry

                lax.fori_loop(0, L // 4, l_body, 0)
                pending_out[sl] = start_out(s, sl)
        for p in pending_out:
            if p is not None:
                p.wait()

    return hist


def _tc_head_body(inv_l, counts_ref, emb_ref, w_ref, b_ref, out_ref):
    m2 = lax.dot_general(emb_ref[...], w_ref[...],
                         (((1,), (1,)), ((), ())),
                         preferred_element_type=jnp.float32) * inv_l
    h = pl.program_id(0)
    m2s = jnp.concatenate([m2, m2], axis=0)
    rid = lax.broadcasted_iota(jnp.int32, (2 * V, V), 0)
    m2h = jnp.where((rid // V) == h, m2s, 0.0)
    out_ref[...] = lax.dot_general(m2h, counts_ref[...],
                                   (((0,), (1,)), ((), ())),
                                   preferred_element_type=jnp.float32) + b_ref[...]


def _tc_head(counts2, emb, W, bcol, L):
    Bh = counts2.shape[0]
    tb = min(Bh, 4096)
    nb = Bh // tb
    return pl.pallas_call(
        functools.partial(_tc_head_body, 1.0 / L),
        grid=(2, nb),
        in_specs=[
            pl.BlockSpec((tb, 2 * V), lambda h, j: (j, 0)),
            pl.BlockSpec((V, D), lambda h, j: (0, 0)),
            pl.BlockSpec((V, D), lambda h, j: (0, 0)),
            pl.BlockSpec((V, 1), lambda h, j: (0, 0)),
        ],
        out_specs=pl.BlockSpec((V, tb), lambda h, j: (0, h * nb + j)),
        out_shape=jax.ShapeDtypeStruct((V, 2 * Bh), jnp.float32),
    )(counts2, emb, W, bcol)


def kernel(x, emb, W, b):
    B, L = x.shape
    x = x.astype(jnp.int32)
    counts2 = _make_sc_histogram(B, L)(x.T).reshape(B // 2, 2 * V)
    out_t = _tc_head(counts2, emb, W, b.reshape(V, 1), L)
    return out_t.T


# speedup vs baseline: 309.7561x; 1.0306x over previous
import functools

import jax
import jax.numpy as jnp
from jax import lax
from jax.experimental import pallas as pl
from jax.experimental.pallas import tpu as pltpu
from jax.experimental.pallas import tpu_sc as plsc

V = 64
D = 16
LANES = 16


def _make_sc_histogram(B, L):
    info = plsc.get_sparse_core_info()
    NC, NS = info.num_cores, info.num_subcores
    NW = NC * NS
    half_rows_w = (B // 2) // NW
    chunkb = min(half_rows_w, 128)
    n_chunks = half_rows_w // chunkb
    subb = chunkb // 2
    n_groups = subb // LANES

    mesh = plsc.VectorSubcoreMesh(core_axis_name="c", subcore_axis_name="s")

    @functools.partial(
        pl.kernel,
        mesh=mesh,
        compiler_params=pltpu.CompilerParams(needs_layout_passes=False),
        out_type=jax.ShapeDtypeStruct(((B // 2) * 2 * V,), jnp.float32),
        scratch_types=[
            pltpu.VMEM((L, chunkb), jnp.int32),
            pltpu.VMEM((L, chunkb), jnp.int32),
            pltpu.VMEM((L, chunkb), jnp.int32),
            pltpu.VMEM((L, chunkb), jnp.int32),
            pltpu.VMEM((subb * 2 * V,), jnp.float32),
            pltpu.VMEM((subb * 2 * V,), jnp.float32),
            pltpu.SemaphoreType.DMA,
            pltpu.SemaphoreType.DMA,
            pltpu.SemaphoreType.DMA,
            pltpu.SemaphoreType.DMA,
        ],
    )
    def hist(xt_hbm, out_hbm, xa0, xa1, xb0, xb1, cbuf0, cbuf1,
             in0, in1, out0, out1):
        wid = lax.axis_index("s") * NC + lax.axis_index("c")
        base2 = wid * half_rows_w
        zeros16 = jnp.zeros((LANES,), jnp.float32)
        ones16 = jnp.full((LANES,), 1.0, jnp.float32)
        row_step = lax.iota(jnp.int32, LANES) * (2 * V)
        xabufs = (xa0, xa1)
        xbbufs = (xb0, xb1)
        cbufs = (cbuf0, cbuf1)
        in_sems = (in0, in1)
        out_sems = (out0, out1)

        def start_in(c):
            col0 = base2 + c * chunkb
            sem = in_sems[c % 2]
            return [
                pltpu.async_copy(
                    xt_hbm.at[:, pl.ds(col0, chunkb)], xabufs[c % 2], sem),
                pltpu.async_copy(
                    xt_hbm.at[:, pl.ds(col0 + B // 2, chunkb)],
                    xbbufs[c % 2], sem),
            ]

        def start_out(s, sl):
            row0 = base2 + s * subb
            return pltpu.async_copy(
                cbufs[sl],
                out_hbm.at[pl.ds(row0 * 2 * V, subb * 2 * V)],
                out_sems[sl])

        pending_in = [start_in(c) for c in range(n_chunks)]
        pending_out = [None, None]
        for c in range(n_chunks):
            for h in pending_in[c]:
                h.wait()
            xa = xabufs[c % 2]
            xb = xbbufs[c % 2]
            for sub in range(chunkb // subb):
                s = c * (chunkb // subb) + sub
                sl = s % 2
                if pending_out[sl] is not None:
                    pending_out[sl].wait()
                    pending_out[sl] = None
                cb = cbufs[sl]

                @plsc.parallel_loop(0, subb * 2 * V // LANES, 1, unroll=8)
                def zero_body(r):
                    cb[pl.ds(r * LANES, LANES)] = zeros16

                def l_body(li, carry):
                    work = []
                    for u in range(4):
                        l = li * 4 + u
                        for p, xp in ((0, xa), (1, xb)):
                            for g in range(n_groups):
                                win = cb.at[pl.ds(
                                    g * LANES * 2 * V + p * V,
                                    (LANES - 1) * 2 * V + V)]
                                tok = xp[l, pl.ds(sub * subb + g * LANES,
                                                  LANES)]
                                work.append((win, tok))
                    idxs = [row_step + tok for _, tok in work]
                    for (win, _), idx in zip(work, idxs):
                        plsc.addupdate_scatter(win, [idx], ones16)
                    return carry

                lax.fori_loop(0, L // 4, l_body, 0)
                pending_out[sl] = start_out(s, sl)
        for p in pending_out:
            if p is not None:
                p.wait()

    return hist


def _tc_head_body(inv_l, counts_ref, emb_ref, w_ref, b_ref, out_ref):
    m2 = lax.dot_general(emb_ref[...], w_ref[...],
                         (((1,), (1,)), ((), ())),
                         preferred_element_type=jnp.float32) * inv_l
    h = pl.program_id(0)
    m2s = jnp.concatenate([m2, m2], axis=0)
    rid = lax.broadcasted_iota(jnp.int32, (2 * V, V), 0)
    m2h = jnp.where((rid // V) == h, m2s, 0.0)
    out_ref[...] = lax.dot_general(m2h, counts_ref[...],
                                   (((0,), (1,)), ((), ())),
                                   preferred_element_type=jnp.float32) + b_ref[...]


def _tc_head(counts2, emb, W, bcol, L):
    Bh = counts2.shape[0]
    tb = min(Bh, 8192)
    nb = Bh // tb
    return pl.pallas_call(
        functools.partial(_tc_head_body, 1.0 / L),
        grid=(2, nb),
        in_specs=[
            pl.BlockSpec((tb, 2 * V), lambda h, j: (j, 0)),
            pl.BlockSpec((V, D), lambda h, j: (0, 0)),
            pl.BlockSpec((V, D), lambda h, j: (0, 0)),
            pl.BlockSpec((V, 1), lambda h, j: (0, 0)),
        ],
        out_specs=pl.BlockSpec((V, tb), lambda h, j: (0, h * nb + j)),
        out_shape=jax.ShapeDtypeStruct((V, 2 * Bh), jnp.float32),
    )(counts2, emb, W, bcol)


def kernel(x, emb, W, b):
    B, L = x.shape
    x = x.astype(jnp.int32)
    counts2 = _make_sc_histogram(B, L)(x.T).reshape(B // 2, 2 * V)
    out_t = _tc_head(counts2, emb, W, b.reshape(V, 1), L)
    return out_t.T


# speedup vs baseline: 325.3238x; 1.0503x over previous
import functools

import jax
import jax.numpy as jnp
from jax import lax
from jax.experimental import pallas as pl
from jax.experimental.pallas import tpu as pltpu
from jax.experimental.pallas import tpu_sc as plsc

V = 64
D = 16
LANES = 16


def _make_sc_histogram(B, L):
    info = plsc.get_sparse_core_info()
    NC, NS = info.num_cores, info.num_subcores
    NW = NC * NS
    half_rows_w = (B // 2) // NW
    chunkb = min(half_rows_w, 128)
    n_chunks = half_rows_w // chunkb
    subb = chunkb // 2
    n_groups = subb // LANES

    mesh = plsc.VectorSubcoreMesh(core_axis_name="c", subcore_axis_name="s")

    @functools.partial(
        pl.kernel,
        mesh=mesh,
        compiler_params=pltpu.CompilerParams(needs_layout_passes=False),
        out_type=jax.ShapeDtypeStruct(((B // 2) * 2 * V,), jnp.float32),
        scratch_types=[
            pltpu.VMEM((L, chunkb), jnp.int32),
            pltpu.VMEM((L, chunkb), jnp.int32),
            pltpu.VMEM((L, chunkb), jnp.int32),
            pltpu.VMEM((L, chunkb), jnp.int32),
            pltpu.VMEM((subb * 2 * V,), jnp.float32),
            pltpu.VMEM((subb * 2 * V,), jnp.float32),
            pltpu.SemaphoreType.DMA,
            pltpu.SemaphoreType.DMA,
            pltpu.SemaphoreType.DMA,
            pltpu.SemaphoreType.DMA,
        ],
    )
    def hist(xt_hbm, out_hbm, xa0, xa1, xb0, xb1, cbuf0, cbuf1,
             in0, in1, out0, out1):
        wid = lax.axis_index("s") * NC + lax.axis_index("c")
        base2 = wid * half_rows_w
        zeros16 = jnp.zeros((LANES,), jnp.float32)
        ones16 = jnp.full((LANES,), 1.0, jnp.float32)
        row_step = lax.iota(jnp.int32, LANES) * (2 * V)
        xabufs = (xa0, xa1)
        xbbufs = (xb0, xb1)
        cbufs = (cbuf0, cbuf1)
        in_sems = (in0, in1)
        out_sems = (out0, out1)

        def start_in(c):
            col0 = base2 + c * chunkb
            sem = in_sems[c % 2]
            return [
                pltpu.async_copy(
                    xt_hbm.at[:, pl.ds(col0, chunkb)], xabufs[c % 2], sem),
                pltpu.async_copy(
                    xt_hbm.at[:, pl.ds(col0 + B // 2, chunkb)],
                    xbbufs[c % 2], sem),
            ]

        def start_out(s, sl):
            row0 = base2 + s * subb
            return pltpu.async_copy(
                cbufs[sl],
                out_hbm.at[pl.ds(row0 * 2 * V, subb * 2 * V)],
                out_sems[sl])

        pending_in = [start_in(0)]
        pending_out = [None, None]
        for c in range(n_chunks):
            for h in pending_in[c]:
                h.wait()
            if c + 1 < n_chunks:
                pending_in.append(start_in(c + 1))
            xa = xabufs[c % 2]
            xb = xbbufs[c % 2]
            for sub in range(chunkb // subb):
                s = c * (chunkb // subb) + sub
                sl = s % 2
                if pending_out[sl] is not None:
                    pending_out[sl].wait()
                    pending_out[sl] = None
                cb = cbufs[sl]

                @plsc.parallel_loop(0, subb * 2 * V // LANES, 1, unroll=8)
                def zero_body(r):
                    cb[pl.ds(r * LANES, LANES)] = zeros16

                def l_body(li, carry):
                    work = []
                    for u in range(4):
                        l = li * 4 + u
                        for p, xp in ((0, xa), (1, xb)):
                            for g in range(n_groups):
                                win = cb.at[pl.ds(
                                    g * LANES * 2 * V + p * V,
                                    (LANES - 1) * 2 * V + V)]
                                tok = xp[l, pl.ds(sub * subb + g * LANES,
                                                  LANES)]
                                work.append((win, tok))
                    idxs = [row_step + tok for _, tok in work]
                    for (win, _), idx in zip(work, idxs):
                        plsc.addupdate_scatter(win, [idx], ones16)
                    return carry

                lax.fori_loop(0, L // 4, l_body, 0)
                pending_out[sl] = start_out(s, sl)
        for p in pending_out:
            if p is not None:
                p.wait()

    return hist


def _tc_head_body(inv_l, counts_ref, emb_ref, w_ref, b_ref, out_ref):
    m2 = lax.dot_general(emb_ref[...], w_ref[...],
                         (((1,), (1,)), ((), ())),
                         preferred_element_type=jnp.float32) * inv_l
    h = pl.program_id(0)
    m2s = jnp.concatenate([m2, m2], axis=0)
    rid = lax.broadcasted_iota(jnp.int32, (2 * V, V), 0)
    m2h = jnp.where((rid // V) == h, m2s, 0.0)
    out_ref[...] = lax.dot_general(m2h, counts_ref[...],
                                   (((0,), (1,)), ((), ())),
                                   preferred_element_type=jnp.float32) + b_ref[...]


def _tc_head(counts2, emb, W, bcol, L):
    Bh = counts2.shape[0]
    tb = min(Bh, 8192)
    nb = Bh // tb
    return pl.pallas_call(
        functools.partial(_tc_head_body, 1.0 / L),
        grid=(2, nb),
        in_specs=[
            pl.BlockSpec((tb, 2 * V), lambda h, j: (j, 0)),
            pl.BlockSpec((V, D), lambda h, j: (0, 0)),
            pl.BlockSpec((V, D), lambda h, j: (0, 0)),
            pl.BlockSpec((V, 1), lambda h, j: (0, 0)),
        ],
        out_specs=pl.BlockSpec((V, tb), lambda h, j: (0, h * nb + j)),
        out_shape=jax.ShapeDtypeStruct((V, 2 * Bh), jnp.float32),
    )(counts2, emb, W, bcol)


def kernel(x, emb, W, b):
    B, L = x.shape
    x = x.astype(jnp.int32)
    counts2 = _make_sc_histogram(B, L)(x.T).reshape(B // 2, 2 * V)
    out_t = _tc_head(counts2, emb, W, b.reshape(V, 1), L)
    return out_t.T
